# baseline (speedup 1.0000x reference)
import functools

import jax
import jax.numpy as jnp
from jax import lax
from jax.experimental import pallas as pl
from jax.experimental.pallas import tpu as pltpu
from jax.experimental.pallas import tpu_sc as plsc

NC, NS = 2, 16
NWORK = NC * NS
_SC_MESH = dict(core_axis_name="c", subcore_axis_name="s",
                num_cores=NC, num_subcores=NS)


def _ln_relu(y, g, b):
    mu = jnp.mean(y, axis=-1, keepdims=True)
    var = jnp.mean((y - mu) ** 2, axis=-1, keepdims=True)
    return jax.nn.relu((y - mu) / jnp.sqrt(var + 1e-5) * g + b)


def _matmul_bias(x, wt, b, blk=2000):
    R, K = x.shape
    N = wt.shape[1]

    def body(x_ref, wt_ref, b_ref, o_ref):
        o_ref[...] = jnp.dot(x_ref[...], wt_ref[...],
                             preferred_element_type=jnp.float32) + b_ref[...]

    return pl.pallas_call(
        body,
        grid=(R // blk,),
        in_specs=[pl.BlockSpec((blk, K), lambda i: (i, 0)),
                  pl.BlockSpec((K, N), lambda i: (0, 0)),
                  pl.BlockSpec((1, N), lambda i: (0, 0))],
        out_specs=pl.BlockSpec((blk, N), lambda i: (i, 0)),
        out_shape=jax.ShapeDtypeStruct((R, N), jnp.float32),
    )(x, wt, b)


def _rnn(g_seq, h0p, lenp, whh_t, bhh, blk=2048):
    Mt, Wp, D = g_seq.shape

    def body(g_ref, h0_ref, len_ref, whh_ref, bhh_ref, out_ref, h_scr):
        t = pl.program_id(1)

        @pl.when(t == 0)
        def _():
            h_scr[...] = h0_ref[...]

        h = h_scr[...]
        hn = jnp.tanh(g_ref[0] + jnp.dot(h, whh_ref[...],
                                         preferred_element_type=jnp.float32)
                      + bhh_ref[...])
        mask = t < len_ref[...]
        h_scr[...] = jnp.where(mask, hn, h)

        @pl.when(t == Mt - 1)
        def _():
            out_ref[...] = h_scr[...]

    return pl.pallas_call(
        body,
        grid=(Wp // blk, Mt),
        in_specs=[pl.BlockSpec((1, blk, D), lambda w, t: (t, w, 0)),
                  pl.BlockSpec((blk, D), lambda w, t: (w, 0)),
                  pl.BlockSpec((blk, 1), lambda w, t: (w, 0)),
                  pl.BlockSpec((D, D), lambda w, t: (0, 0)),
                  pl.BlockSpec((1, D), lambda w, t: (0, 0))],
        out_specs=pl.BlockSpec((blk, D), lambda w, t: (w, 0)),
        out_shape=jax.ShapeDtypeStruct((Wp, D), jnp.float32),
        scratch_shapes=[pltpu.VMEM((blk, D), jnp.float32)],
        compiler_params=pltpu.CompilerParams(
            dimension_semantics=("parallel", "arbitrary")),
    )(g_seq, h0p, lenp, whh_t, bhh)


def _mlp(xp, w1t, g1, be1, w2t, g2, be2, w3t, b3, n_valid, blk=2048):
    Fp, K = xp.shape
    H = w3t.shape[1]

    def body(x_ref, w1_ref, g1_ref, be1_ref, w2_ref, g2_ref, be2_ref,
             w3_ref, b3_ref, o_ref):
        i = pl.program_id(0)
        h1 = _ln_relu(jnp.dot(x_ref[...], w1_ref[...],
                              preferred_element_type=jnp.float32),
                      g1_ref[...], be1_ref[...])
        h2 = _ln_relu(jnp.dot(h1, w2_ref[...],
                              preferred_element_type=jnp.float32),
                      g2_ref[...], be2_ref[...])
        p = jnp.dot(h2, w3_ref[...],
                    preferred_element_type=jnp.float32) + b3_ref[...]
        rows = i * blk + lax.broadcasted_iota(jnp.int32, (blk, 1), 0)
        o_ref[...] = jnp.where(rows < n_valid, p, 0.0)

    return pl.pallas_call(
        body,
        grid=(Fp // blk,),
        in_specs=[pl.BlockSpec((blk, K), lambda i: (i, 0)),
                  pl.BlockSpec((K, 512), lambda i: (0, 0)),
                  pl.BlockSpec((1, 512), lambda i: (0, 0)),
                  pl.BlockSpec((1, 512), lambda i: (0, 0)),
                  pl.BlockSpec((512, 256), lambda i: (0, 0)),
                  pl.BlockSpec((1, 256), lambda i: (0, 0)),
                  pl.BlockSpec((1, 256), lambda i: (0, 0)),
                  pl.BlockSpec((256, H), lambda i: (0, 0)),
                  pl.BlockSpec((1, H), lambda i: (0, 0))],
        out_specs=pl.BlockSpec((blk, H), lambda i: (i, 0)),
        out_shape=jax.ShapeDtypeStruct((Fp, H), jnp.float32),
    )(xp, w1t, g1, be1, w2t, g2, be2, w3t, b3)


def _final(ff, s2, w4at, w4bt, g4, be4, blk=2000):
    R, K = ff.shape
    D = w4at.shape[1]

    def body(a_ref, b_ref, wa_ref, wb_ref, g_ref, be_ref, o_ref):
        y = (jnp.dot(a_ref[...], wa_ref[...],
                     preferred_element_type=jnp.float32)
             + jnp.dot(b_ref[...], wb_ref[...],
                       preferred_element_type=jnp.float32))
        o_ref[...] = _ln_relu(y, g_ref[...], be_ref[...])

    return pl.pallas_call(
        body,
        grid=(R // blk,),
        in_specs=[pl.BlockSpec((blk, K), lambda i: (i, 0)),
                  pl.BlockSpec((blk, K), lambda i: (i, 0)),
                  pl.BlockSpec((K, D), lambda i: (0, 0)),
                  pl.BlockSpec((K, D), lambda i: (0, 0)),
                  pl.BlockSpec((1, D), lambda i: (0, 0)),
                  pl.BlockSpec((1, D), lambda i: (0, 0))],
        out_specs=pl.BlockSpec((blk, D), lambda i: (i, 0)),
        out_shape=jax.ShapeDtypeStruct((R, D), jnp.float32),
    )(ff, s2, w4at, w4bt, g4, be4)


def _sc_gather(table, idx, CH=128):
    B = idx.shape[0]
    T, D = table.shape
    per_w = B // NWORK
    n_ch = per_w // CH
    mesh = plsc.VectorSubcoreMesh(**_SC_MESH)

    @functools.partial(
        pl.kernel, mesh=mesh,
        out_type=jax.ShapeDtypeStruct((B, D), jnp.float32),
        scratch_types=[pltpu.VMEM((CH,), jnp.int32),
                       pltpu.VMEM((CH, D), jnp.float32)],
    )
    def k(table_hbm, idx_hbm, out_hbm, idx_v, rows_v):
        wid = lax.axis_index("s") * NC + lax.axis_index("c")
        base = wid * per_w

        def step(g, carry):
            off = base + g * CH
            pltpu.sync_copy(idx_hbm.at[pl.ds(off, CH)], idx_v)
            pltpu.sync_copy(table_hbm.at[idx_v], rows_v)
            pltpu.sync_copy(rows_v, out_hbm.at[pl.ds(off, CH)])
            return carry

        lax.fori_loop(0, n_ch, step, 0)

    return k(table, idx)


def _sc_gather_sum(table, idx, lens, zero_row, GRP=8):
    Tp, D = table.shape
    Fp, M = idx.shape
    per_w = Fp // NWORK
    n_g = per_w // GRP
    NIDX = GRP * M
    mesh = plsc.VectorSubcoreMesh(**_SC_MESH)

    @functools.partial(
        pl.kernel, mesh=mesh,
        out_type=jax.ShapeDtypeStruct((Fp, D), jnp.float32),
        scratch_types=[pltpu.VMEM((per_w + 16,), jnp.int32),
                       pltpu.VMEM((NIDX,), jnp.int32),
                       pltpu.VMEM((NIDX, D), jnp.float32),
                       pltpu.VMEM((GRP, D), jnp.float32)],
    )
    def k(table_hbm, idx_hbm, len_hbm, out_hbm, len_v, idx_v, rows_v, acc_v):
        wid = lax.axis_index("s") * NC + lax.axis_index("c")
        fbase = wid * per_w
        pltpu.sync_copy(len_hbm.at[pl.ds(fbase, per_w)],
                        len_v.at[pl.ds(0, per_w)])
        lane = lax.broadcasted_iota(jnp.int32, (16,), 0)

        def group(g, carry):
            pltpu.sync_copy(idx_hbm.at[pl.ds((fbase + g * GRP) * M, NIDX)],
                            idx_v)
            lv = len_v[pl.ds(g * GRP, 16)]
            for j in range(GRP):
                fl = jnp.full((16,), lv[j], jnp.int32)
                raw = idx_v[pl.ds(j * M, 16)]
                idx_v[pl.ds(j * M, 16)] = jnp.where(lane < fl, raw, zero_row)
            pltpu.sync_copy(table_hbm.at[idx_v], rows_v)
            for j in range(GRP):
                for db in range(D // 16):
                    s = rows_v[j * M, pl.ds(db * 16, 16)]
                    for m in range(1, M):
                        s = s + rows_v[j * M + m, pl.ds(db * 16, 16)]
                    acc_v[j, pl.ds(db * 16, 16)] = s
            pltpu.sync_copy(acc_v, out_hbm.at[pl.ds(fbase + g * GRP, GRP)])
            return carry

        lax.fori_loop(0, n_g, group, 0)

    return k(table, idx.reshape(-1), lens)


def _pad_rows(x, n):
    return jnp.pad(x, ((0, n - x.shape[0]),) + ((0, 0),) * (x.ndim - 1))


def kernel(edges, faces, edge_index, wire_index, face_index,
           edge_index_length, wire_index_length, adj_face_index_length,
           h0, W_ih, W_hh, b_ih, b_hh, W1, g1, be1, W2, g2, be2, W3, b3,
           W4, g4, be4):
    NE, D = edges.shape
    NF = faces.shape[0]
    NW, Mt = edge_index.shape[0], edge_index.shape[1]
    PAD = NWORK * 8 * 40
    Wp = PAD
    Fp = PAD

    eproj = _matmul_bias(edges, W_ih.T, b_ih.reshape(1, -1))
    eidx = _pad_rows(edge_index, Wp).T.reshape(-1)
    g_seq = _sc_gather(eproj, eidx).reshape(Mt, Wp, D)
    h0p = _pad_rows(h0, Wp)
    elenp = _pad_rows(edge_index_length.reshape(-1, 1), Wp)
    feat_wire = _rnn(g_seq, h0p, elenp, W_hh.T, b_hh.reshape(1, -1))

    widx = _pad_rows(wire_index, Fp)
    wlenp = _pad_rows(wire_index_length.reshape(-1, 1), Fp).reshape(-1)
    s1 = _sc_gather_sum(feat_wire, widx, wlenp, zero_row=NW)[:NF]

    ff = jnp.concatenate([faces, s1], axis=-1)

    p = _mlp(_pad_rows(ff, Fp), W1.T, g1.reshape(1, -1), be1.reshape(1, -1),
             W2.T, g2.reshape(1, -1), be2.reshape(1, -1),
             W3.T, b3.reshape(1, -1), n_valid=NF)

    fidx = _pad_rows(face_index, Fp)
    alenp = _pad_rows(adj_face_index_length.reshape(-1, 1), Fp).reshape(-1)
    s2 = _sc_gather_sum(p, fidx, alenp, zero_row=NF)[:NF]

    return _final(ff, s2, W4[:, :2 * D].T, W4[:, 2 * D:].T,
                  g4.reshape(1, -1), be4.reshape(1, -1))


# speedup vs baseline: 5.6890x; 5.6890x over previous
import functools

import jax
import jax.numpy as jnp
from jax import lax
from jax.experimental import pallas as pl
from jax.experimental.pallas import tpu as pltpu
from jax.experimental.pallas import tpu_sc as plsc

NC, NS = 2, 16
NWORK = NC * NS
_SC_MESH = dict(core_axis_name="c", subcore_axis_name="s",
                num_cores=NC, num_subcores=NS)


def _ln_relu(y, g, b):
    mu = jnp.mean(y, axis=-1, keepdims=True)
    var = jnp.mean((y - mu) ** 2, axis=-1, keepdims=True)
    return jax.nn.relu((y - mu) / jnp.sqrt(var + 1e-5) * g + b)


def _matmul_bias(x, wt, b, blk=2000):
    R, K = x.shape
    N = wt.shape[1]

    def body(x_ref, wt_ref, b_ref, o_ref):
        o_ref[...] = jnp.dot(x_ref[...], wt_ref[...],
                             preferred_element_type=jnp.float32) + b_ref[...]

    return pl.pallas_call(
        body,
        grid=(R // blk,),
        in_specs=[pl.BlockSpec((blk, K), lambda i: (i, 0)),
                  pl.BlockSpec((K, N), lambda i: (0, 0)),
                  pl.BlockSpec((1, N), lambda i: (0, 0))],
        out_specs=pl.BlockSpec((blk, N), lambda i: (i, 0)),
        out_shape=jax.ShapeDtypeStruct((R, N), jnp.float32),
    )(x, wt, b)


def _rnn(g_seq, h0p, lenp, whh_t, bhh, blk=2048):
    Mt, Wp, D = g_seq.shape

    def body(g_ref, h0_ref, len_ref, whh_ref, bhh_ref, out_ref, h_scr):
        t = pl.program_id(1)

        @pl.when(t == 0)
        def _():
            h_scr[...] = h0_ref[...]

        h = h_scr[...]
        hn = jnp.tanh(g_ref[0] + jnp.dot(h, whh_ref[...],
                                         preferred_element_type=jnp.float32)
                      + bhh_ref[...])
        mask = t < len_ref[...]
        h_scr[...] = jnp.where(mask, hn, h)

        @pl.when(t == Mt - 1)
        def _():
            out_ref[...] = h_scr[...]

    return pl.pallas_call(
        body,
        grid=(Wp // blk, Mt),
        in_specs=[pl.BlockSpec((1, blk, D), lambda w, t: (t, w, 0)),
                  pl.BlockSpec((blk, D), lambda w, t: (w, 0)),
                  pl.BlockSpec((blk, 1), lambda w, t: (w, 0)),
                  pl.BlockSpec((D, D), lambda w, t: (0, 0)),
                  pl.BlockSpec((1, D), lambda w, t: (0, 0))],
        out_specs=pl.BlockSpec((blk, D), lambda w, t: (w, 0)),
        out_shape=jax.ShapeDtypeStruct((Wp, D), jnp.float32),
        scratch_shapes=[pltpu.VMEM((blk, D), jnp.float32)],
        compiler_params=pltpu.CompilerParams(
            dimension_semantics=("parallel", "arbitrary")),
    )(g_seq, h0p, lenp, whh_t, bhh)


def _mlp(xp, w1t, g1, be1, w2t, g2, be2, w3t, b3, n_valid, blk=2048):
    Fp, K = xp.shape
    H = w3t.shape[1]

    def body(x_ref, w1_ref, g1_ref, be1_ref, w2_ref, g2_ref, be2_ref,
             w3_ref, b3_ref, o_ref):
        i = pl.program_id(0)
        h1 = _ln_relu(jnp.dot(x_ref[...], w1_ref[...],
                              preferred_element_type=jnp.float32),
                      g1_ref[...], be1_ref[...])
        h2 = _ln_relu(jnp.dot(h1, w2_ref[...],
                              preferred_element_type=jnp.float32),
                      g2_ref[...], be2_ref[...])
        p = jnp.dot(h2, w3_ref[...],
                    preferred_element_type=jnp.float32) + b3_ref[...]
        rows = i * blk + lax.broadcasted_iota(jnp.int32, (blk, 1), 0)
        o_ref[...] = jnp.where(rows < n_valid, p, 0.0)

    return pl.pallas_call(
        body,
        grid=(Fp // blk,),
        in_specs=[pl.BlockSpec((blk, K), lambda i: (i, 0)),
                  pl.BlockSpec((K, 512), lambda i: (0, 0)),
                  pl.BlockSpec((1, 512), lambda i: (0, 0)),
                  pl.BlockSpec((1, 512), lambda i: (0, 0)),
                  pl.BlockSpec((512, 256), lambda i: (0, 0)),
                  pl.BlockSpec((1, 256), lambda i: (0, 0)),
                  pl.BlockSpec((1, 256), lambda i: (0, 0)),
                  pl.BlockSpec((256, H), lambda i: (0, 0)),
                  pl.BlockSpec((1, H), lambda i: (0, 0))],
        out_specs=pl.BlockSpec((blk, H), lambda i: (i, 0)),
        out_shape=jax.ShapeDtypeStruct((Fp, H), jnp.float32),
    )(xp, w1t, g1, be1, w2t, g2, be2, w3t, b3)


def _final(ff, s2, w4at, w4bt, g4, be4, blk=2000):
    R, K = ff.shape
    D = w4at.shape[1]

    def body(a_ref, b_ref, wa_ref, wb_ref, g_ref, be_ref, o_ref):
        y = (jnp.dot(a_ref[...], wa_ref[...],
                     preferred_element_type=jnp.float32)
             + jnp.dot(b_ref[...], wb_ref[...],
                       preferred_element_type=jnp.float32))
        o_ref[...] = _ln_relu(y, g_ref[...], be_ref[...])

    return pl.pallas_call(
        body,
        grid=(R // blk,),
        in_specs=[pl.BlockSpec((blk, K), lambda i: (i, 0)),
                  pl.BlockSpec((blk, K), lambda i: (i, 0)),
                  pl.BlockSpec((K, D), lambda i: (0, 0)),
                  pl.BlockSpec((K, D), lambda i: (0, 0)),
                  pl.BlockSpec((1, D), lambda i: (0, 0)),
                  pl.BlockSpec((1, D), lambda i: (0, 0))],
        out_specs=pl.BlockSpec((blk, D), lambda i: (i, 0)),
        out_shape=jax.ShapeDtypeStruct((R, D), jnp.float32),
    )(ff, s2, w4at, w4bt, g4, be4)


def _sc_gather(table, idx, CH=128):
    B = idx.shape[0]
    T, D = table.shape
    per_w = B // NWORK
    n_ch = per_w // CH
    mesh = plsc.VectorSubcoreMesh(**_SC_MESH)

    @functools.partial(
        pl.kernel, mesh=mesh,
        out_type=jax.ShapeDtypeStruct((B, D), jnp.float32),
        scratch_types=[pltpu.VMEM((CH,), jnp.int32),
                       pltpu.VMEM((CH, D), jnp.float32)],
    )
    def k(table_hbm, idx_hbm, out_hbm, idx_v, rows_v):
        wid = lax.axis_index("s") * NC + lax.axis_index("c")
        base = wid * per_w

        def step(g, carry):
            off = base + g * CH
            pltpu.sync_copy(idx_hbm.at[pl.ds(off, CH)], idx_v)
            pltpu.sync_copy(table_hbm.at[idx_v], rows_v)
            pltpu.sync_copy(rows_v, out_hbm.at[pl.ds(off, CH)])
            return carry

        lax.fori_loop(0, n_ch, step, 0)

    return k(table, idx)


def _sc_gather_sum(table, idx, lens, zero_row, spread=2048, GRP=8):
    Tp, D = table.shape
    Fp, M = idx.shape
    per_w = Fp // NWORK
    n_g = per_w // GRP
    NIDX = GRP * M
    mesh = plsc.VectorSubcoreMesh(**_SC_MESH)

    @functools.partial(
        pl.kernel, mesh=mesh,
        out_type=jax.ShapeDtypeStruct((Fp, D), jnp.float32),
        scratch_types=[pltpu.VMEM((per_w + 16,), jnp.int32),
                       pltpu.VMEM((NIDX,), jnp.int32),
                       pltpu.VMEM((NIDX, D), jnp.float32),
                       pltpu.VMEM((GRP, D), jnp.float32)],
    )
    def k(table_hbm, idx_hbm, len_hbm, out_hbm, len_v, idx_v, rows_v, acc_v):
        wid = lax.axis_index("s") * NC + lax.axis_index("c")
        fbase = wid * per_w
        pltpu.sync_copy(len_hbm.at[pl.ds(fbase, per_w)],
                        len_v.at[pl.ds(0, per_w)])
        lane = lax.broadcasted_iota(jnp.int32, (16,), 0)

        def group(g, carry):
            pltpu.sync_copy(idx_hbm.at[pl.ds((fbase + g * GRP) * M, NIDX)],
                            idx_v)
            lv = len_v[pl.ds(g * GRP, 16)]
            for j in range(GRP):
                fl = jnp.full((16,), lv[j], jnp.int32)
                raw = idx_v[pl.ds(j * M, 16)]
                pad_row = zero_row + (raw & (spread - 1))
                idx_v[pl.ds(j * M, 16)] = jnp.where(lane < fl, raw, pad_row)
            pltpu.sync_copy(table_hbm.at[idx_v], rows_v)
            for j in range(GRP):
                for db in range(D // 16):
                    s = rows_v[j * M, pl.ds(db * 16, 16)]
                    for m in range(1, M):
                        s = s + rows_v[j * M + m, pl.ds(db * 16, 16)]
                    acc_v[j, pl.ds(db * 16, 16)] = s
            pltpu.sync_copy(acc_v, out_hbm.at[pl.ds(fbase + g * GRP, GRP)])
            return carry

        lax.fori_loop(0, n_g, group, 0)

    return k(table, idx.reshape(-1), lens)


def _pad_rows(x, n):
    return jnp.pad(x, ((0, n - x.shape[0]),) + ((0, 0),) * (x.ndim - 1))


def kernel(edges, faces, edge_index, wire_index, face_index,
           edge_index_length, wire_index_length, adj_face_index_length,
           h0, W_ih, W_hh, b_ih, b_hh, W1, g1, be1, W2, g2, be2, W3, b3,
           W4, g4, be4):
    NE, D = edges.shape
    NF = faces.shape[0]
    NW, Mt = edge_index.shape[0], edge_index.shape[1]
    PAD = NWORK * 8 * 40
    Wp = PAD
    Fp = PAD

    eproj = _matmul_bias(edges, W_ih.T, b_ih.reshape(1, -1))
    eidx = _pad_rows(edge_index, Wp).T.reshape(-1)
    g_seq = _sc_gather(eproj, eidx).reshape(Mt, Wp, D)
    h0p = _pad_rows(h0, Wp)
    elenp = _pad_rows(edge_index_length.reshape(-1, 1), Wp)
    feat_wire = _rnn(g_seq, h0p, elenp, W_hh.T, b_hh.reshape(1, -1))

    SPREAD = 2048
    Tp = PAD + SPREAD
    widx = _pad_rows(wire_index, Fp)
    wlenp = _pad_rows(wire_index_length.reshape(-1, 1), Fp).reshape(-1)
    s1 = _sc_gather_sum(_pad_rows(feat_wire, Tp), widx, wlenp,
                        zero_row=NW, spread=SPREAD)[:NF]

    ff = jnp.concatenate([faces, s1], axis=-1)

    p = _mlp(_pad_rows(ff, Fp), W1.T, g1.reshape(1, -1), be1.reshape(1, -1),
             W2.T, g2.reshape(1, -1), be2.reshape(1, -1),
             W3.T, b3.reshape(1, -1), n_valid=NF)

    fidx = _pad_rows(face_index, Fp)
    alenp = _pad_rows(adj_face_index_length.reshape(-1, 1), Fp).reshape(-1)
    s2 = _sc_gather_sum(_pad_rows(p, Tp), fidx, alenp,
                        zero_row=NF, spread=SPREAD)[:NF]

    return _final(ff, s2, W4[:, :2 * D].T, W4[:, 2 * D:].T,
                  g4.reshape(1, -1), be4.reshape(1, -1))


# speedup vs baseline: 6.6532x; 1.1695x over previous
import functools

import jax
import jax.numpy as jnp
from jax import lax
from jax.experimental import pallas as pl
from jax.experimental.pallas import tpu as pltpu
from jax.experimental.pallas import tpu_sc as plsc

NC, NS = 2, 16
NWORK = NC * NS
_SC_MESH = dict(core_axis_name="c", subcore_axis_name="s",
                num_cores=NC, num_subcores=NS)


def _ln_relu(y, g, b):
    mu = jnp.mean(y, axis=-1, keepdims=True)
    var = jnp.mean((y - mu) ** 2, axis=-1, keepdims=True)
    return jax.nn.relu((y - mu) / jnp.sqrt(var + 1e-5) * g + b)


def _matmul_bias(x, wt, b, blk=2000):
    R, K = x.shape
    N = wt.shape[1]

    def body(x_ref, wt_ref, b_ref, o_ref):
        o_ref[...] = jnp.dot(x_ref[...], wt_ref[...],
                             preferred_element_type=jnp.float32) + b_ref[...]

    return pl.pallas_call(
        body,
        grid=(R // blk,),
        in_specs=[pl.BlockSpec((blk, K), lambda i: (i, 0)),
                  pl.BlockSpec((K, N), lambda i: (0, 0)),
                  pl.BlockSpec((1, N), lambda i: (0, 0))],
        out_specs=pl.BlockSpec((blk, N), lambda i: (i, 0)),
        out_shape=jax.ShapeDtypeStruct((R, N), jnp.float32),
    )(x, wt, b)


def _rnn(g_seq, h0p, lenp, whh_t, bhh, blk=2048):
    Mt, Wp, D = g_seq.shape

    def body(g_ref, h0_ref, len_ref, whh_ref, bhh_ref, out_ref, h_scr):
        t = pl.program_id(1)

        @pl.when(t == 0)
        def _():
            h_scr[...] = h0_ref[...]

        h = h_scr[...]
        hn = jnp.tanh(g_ref[0] + jnp.dot(h, whh_ref[...],
                                         preferred_element_type=jnp.float32)
                      + bhh_ref[...])
        mask = t < len_ref[...]
        h_scr[...] = jnp.where(mask, hn, h)

        @pl.when(t == Mt - 1)
        def _():
            out_ref[...] = h_scr[...]

    return pl.pallas_call(
        body,
        grid=(Wp // blk, Mt),
        in_specs=[pl.BlockSpec((1, blk, D), lambda w, t: (t, w, 0)),
                  pl.BlockSpec((blk, D), lambda w, t: (w, 0)),
                  pl.BlockSpec((blk, 1), lambda w, t: (w, 0)),
                  pl.BlockSpec((D, D), lambda w, t: (0, 0)),
                  pl.BlockSpec((1, D), lambda w, t: (0, 0))],
        out_specs=pl.BlockSpec((blk, D), lambda w, t: (w, 0)),
        out_shape=jax.ShapeDtypeStruct((Wp, D), jnp.float32),
        scratch_shapes=[pltpu.VMEM((blk, D), jnp.float32)],
        compiler_params=pltpu.CompilerParams(
            dimension_semantics=("parallel", "arbitrary")),
    )(g_seq, h0p, lenp, whh_t, bhh)


def _mlp(xp, w1t, g1, be1, w2t, g2, be2, w3t, b3, n_valid, blk=2048):
    Fp, K = xp.shape
    H = w3t.shape[1]

    def body(x_ref, w1_ref, g1_ref, be1_ref, w2_ref, g2_ref, be2_ref,
             w3_ref, b3_ref, o_ref):
        i = pl.program_id(0)
        h1 = _ln_relu(jnp.dot(x_ref[...], w1_ref[...],
                              preferred_element_type=jnp.float32),
                      g1_ref[...], be1_ref[...])
        h2 = _ln_relu(jnp.dot(h1, w2_ref[...],
                              preferred_element_type=jnp.float32),
                      g2_ref[...], be2_ref[...])
        p = jnp.dot(h2, w3_ref[...],
                    preferred_element_type=jnp.float32) + b3_ref[...]
        rows = i * blk + lax.broadcasted_iota(jnp.int32, (blk, 1), 0)
        o_ref[...] = jnp.where(rows < n_valid, p, 0.0)

    return pl.pallas_call(
        body,
        grid=(Fp // blk,),
        in_specs=[pl.BlockSpec((blk, K), lambda i: (i, 0)),
                  pl.BlockSpec((K, 512), lambda i: (0, 0)),
                  pl.BlockSpec((1, 512), lambda i: (0, 0)),
                  pl.BlockSpec((1, 512), lambda i: (0, 0)),
                  pl.BlockSpec((512, 256), lambda i: (0, 0)),
                  pl.BlockSpec((1, 256), lambda i: (0, 0)),
                  pl.BlockSpec((1, 256), lambda i: (0, 0)),
                  pl.BlockSpec((256, H), lambda i: (0, 0)),
                  pl.BlockSpec((1, H), lambda i: (0, 0))],
        out_specs=pl.BlockSpec((blk, H), lambda i: (i, 0)),
        out_shape=jax.ShapeDtypeStruct((Fp, H), jnp.float32),
    )(xp, w1t, g1, be1, w2t, g2, be2, w3t, b3)


def _final(ff, s2, w4at, w4bt, g4, be4, blk=2000):
    R, K = ff.shape
    D = w4at.shape[1]

    def body(a_ref, b_ref, wa_ref, wb_ref, g_ref, be_ref, o_ref):
        y = (jnp.dot(a_ref[...], wa_ref[...],
                     preferred_element_type=jnp.float32)
             + jnp.dot(b_ref[...], wb_ref[...],
                       preferred_element_type=jnp.float32))
        o_ref[...] = _ln_relu(y, g_ref[...], be_ref[...])

    return pl.pallas_call(
        body,
        grid=(R // blk,),
        in_specs=[pl.BlockSpec((blk, K), lambda i: (i, 0)),
                  pl.BlockSpec((blk, K), lambda i: (i, 0)),
                  pl.BlockSpec((K, D), lambda i: (0, 0)),
                  pl.BlockSpec((K, D), lambda i: (0, 0)),
                  pl.BlockSpec((1, D), lambda i: (0, 0)),
                  pl.BlockSpec((1, D), lambda i: (0, 0))],
        out_specs=pl.BlockSpec((blk, D), lambda i: (i, 0)),
        out_shape=jax.ShapeDtypeStruct((R, D), jnp.float32),
    )(ff, s2, w4at, w4bt, g4, be4)


def _sc_gather(table, idx, CH=128, NB=4, split=(60, 20)):
    B = idx.shape[0]
    T, D = table.shape
    C0, C1 = split
    assert NS * (C0 + C1) * CH == B and C0 % NB == 0 and C1 % NB == 0
    IDXLEN = C0 * CH
    mesh = plsc.VectorSubcoreMesh(**_SC_MESH)

    @functools.partial(
        pl.kernel, mesh=mesh,
        out_type=jax.ShapeDtypeStruct((B, D), jnp.float32),
        scratch_types=[pltpu.VMEM((IDXLEN,), jnp.int32),
                       pltpu.VMEM((NB, CH, D), jnp.float32),
                       pltpu.SemaphoreType.DMA,
                       pltpu.SemaphoreType.DMA],
    )
    def k(table_hbm, idxp_hbm, out_hbm, idx_v, rows_v, gsem, wsem):
        cid = lax.axis_index("c")
        sid = lax.axis_index("s")
        chunk0 = jnp.where(cid == 0, sid * C0, NS * C0 + sid * C1)
        nblk = jnp.where(cid == 0, C0 // NB, C1 // NB)
        base = chunk0 * CH
        pltpu.sync_copy(idxp_hbm.at[pl.ds(base, IDXLEN)], idx_v)

        def blk(i, carry):
            c0 = i * NB
            gd, wd = [], []
            for b in range(NB):
                src = table_hbm.at[idx_v.at[pl.ds((c0 + b) * CH, CH)]]
                gd.append(pltpu.async_copy(src, rows_v.at[b], gsem))
            for b in range(NB):
                gd[b].wait()
                dst = out_hbm.at[pl.ds(base + (c0 + b) * CH, CH)]
                wd.append(pltpu.async_copy(rows_v.at[b], dst, wsem))
            for b in range(NB):
                wd[b].wait()
            return carry

        lax.fori_loop(0, nblk, blk, 0)

    return k(table, jnp.pad(idx, (0, IDXLEN)))


def _sc_gather_sum(table, idx, lens, zero_row, spread=2048, GRP=8,
                   split_cores=(60, 20)):
    Tp, D = table.shape
    Fp, M = idx.shape
    NIDX = GRP * M
    NB = 4 if D <= 128 else 2
    C0, C1 = split_cores
    assert NS * (C0 + C1) * GRP == Fp and C0 % NB == 0 and C1 % NB == 0
    FLEN = C0 * GRP
    mesh = plsc.VectorSubcoreMesh(**_SC_MESH)

    @functools.partial(
        pl.kernel, mesh=mesh,
        out_type=jax.ShapeDtypeStruct((Fp, D), jnp.float32),
        scratch_types=[pltpu.VMEM((FLEN + 16,), jnp.int32),
                       pltpu.VMEM((FLEN * M,), jnp.int32),
                       pltpu.VMEM((NB, NIDX, D), jnp.float32),
                       pltpu.VMEM((NB, GRP, D), jnp.float32),
                       pltpu.SemaphoreType.DMA,
                       pltpu.SemaphoreType.DMA],
    )
    def k(table_hbm, idx_hbm, len_hbm, out_hbm, len_v, idx_v, rows_v, acc_v,
          gsem, wsem):
        cid = lax.axis_index("c")
        sid = lax.axis_index("s")
        chunk0 = jnp.where(cid == 0, sid * C0, NS * C0 + sid * C1)
        nblk = jnp.where(cid == 0, C0 // NB, C1 // NB)
        fbase = chunk0 * GRP
        pltpu.sync_copy(len_hbm.at[pl.ds(fbase, FLEN)],
                        len_v.at[pl.ds(0, FLEN)])
        pltpu.sync_copy(idx_hbm.at[pl.ds(fbase * M, FLEN * M)], idx_v)
        lane = lax.broadcasted_iota(jnp.int32, (16,), 0)

        def blk(i, carry):
            c0 = i * NB
            gd, wd = [], []
            for b in range(NB):
                c = c0 + b
                lv = len_v[pl.ds(c * GRP, 16)]
                for j in range(GRP):
                    fl = jnp.full((16,), lv[j], jnp.int32)
                    sl = pl.ds(c * NIDX + j * M, 16)
                    raw = idx_v[sl]
                    pad_row = zero_row + (raw & (spread - 1))
                    idx_v[sl] = jnp.where(lane < fl, raw, pad_row)
                src = table_hbm.at[idx_v.at[pl.ds(c * NIDX, NIDX)]]
                gd.append(pltpu.async_copy(src, rows_v.at[b], gsem))
            for b in range(NB):
                gd[b].wait()
                rv = rows_v.at[b]
                av = acc_v.at[b]

                def face(j, carry2):
                    for db in range(D // 16):
                        cs = pl.ds(db * 16, 16)
                        s = rv[j * M, cs]
                        for m in range(1, M):
                            s = s + rv[j * M + m, cs]
                        av[j, cs] = s
                    return carry2

                lax.fori_loop(0, GRP, face, 0)
                dst = out_hbm.at[pl.ds(fbase + (c0 + b) * GRP, GRP)]
                wd.append(pltpu.async_copy(av, dst, wsem))
            for b in range(NB):
                wd[b].wait()
            return carry

        lax.fori_loop(0, nblk, blk, 0)

    return k(table, jnp.pad(idx.reshape(-1), (0, FLEN * M)),
             jnp.pad(lens, (0, FLEN)))


def _pad_rows(x, n):
    return jnp.pad(x, ((0, n - x.shape[0]),) + ((0, 0),) * (x.ndim - 1))


def kernel(edges, faces, edge_index, wire_index, face_index,
           edge_index_length, wire_index_length, adj_face_index_length,
           h0, W_ih, W_hh, b_ih, b_hh, W1, g1, be1, W2, g2, be2, W3, b3,
           W4, g4, be4):
    NE, D = edges.shape
    NF = faces.shape[0]
    NW, Mt = edge_index.shape[0], edge_index.shape[1]
    PAD = NWORK * 8 * 40
    Wp = PAD
    Fp = PAD

    eproj = _matmul_bias(edges, W_ih.T, b_ih.reshape(1, -1))
    eidx = _pad_rows(edge_index, Wp).T.reshape(-1)
    g_seq = _sc_gather(eproj, eidx).reshape(Mt, Wp, D)
    h0p = _pad_rows(h0, Wp)
    elenp = _pad_rows(edge_index_length.reshape(-1, 1), Wp)
    feat_wire = _rnn(g_seq, h0p, elenp, W_hh.T, b_hh.reshape(1, -1))

    SPREAD = 2048
    Tp = PAD + SPREAD
    widx = _pad_rows(wire_index, Fp)
    wlenp = _pad_rows(wire_index_length.reshape(-1, 1), Fp).reshape(-1)
    s1 = _sc_gather_sum(_pad_rows(feat_wire, Tp), widx, wlenp,
                        zero_row=NW, spread=SPREAD)[:NF]

    ff = jnp.concatenate([faces, s1], axis=-1)

    p = _mlp(_pad_rows(ff, Fp), W1.T, g1.reshape(1, -1), be1.reshape(1, -1),
             W2.T, g2.reshape(1, -1), be2.reshape(1, -1),
             W3.T, b3.reshape(1, -1), n_valid=NF)

    fidx = _pad_rows(face_index, Fp)
    alenp = _pad_rows(adj_face_index_length.reshape(-1, 1), Fp).reshape(-1)
    s2 = _sc_gather_sum(_pad_rows(p, Tp), fidx, alenp, zero_row=NF,
                        spread=SPREAD, split_cores=(52, 28))[:NF]

    return _final(ff, s2, W4[:, :2 * D].T, W4[:, 2 * D:].T,
                  g4.reshape(1, -1), be4.reshape(1, -1))


# speedup vs baseline: 7.1080x; 1.0683x over previous
import functools

import jax
import jax.numpy as jnp
from jax import lax
from jax.experimental import pallas as pl
from jax.experimental.pallas import tpu as pltpu
from jax.experimental.pallas import tpu_sc as plsc

NC, NS = 2, 16
NWORK = NC * NS
_SC_MESH = dict(core_axis_name="c", subcore_axis_name="s",
                num_cores=NC, num_subcores=NS)


def _ln_relu(y, g, b):
    mu = jnp.mean(y, axis=-1, keepdims=True)
    var = jnp.mean((y - mu) ** 2, axis=-1, keepdims=True)
    return jax.nn.relu((y - mu) / jnp.sqrt(var + 1e-5) * g + b)


def _matmul_bias(x, wt, b, blk=2000):
    R, K = x.shape
    N = wt.shape[1]

    def body(x_ref, wt_ref, b_ref, o_ref):
        o_ref[...] = jnp.dot(x_ref[...], wt_ref[...],
                             preferred_element_type=jnp.float32) + b_ref[...]

    return pl.pallas_call(
        body,
        grid=(R // blk,),
        in_specs=[pl.BlockSpec((blk, K), lambda i: (i, 0)),
                  pl.BlockSpec((K, N), lambda i: (0, 0)),
                  pl.BlockSpec((1, N), lambda i: (0, 0))],
        out_specs=pl.BlockSpec((blk, N), lambda i: (i, 0)),
        out_shape=jax.ShapeDtypeStruct((R, N), jnp.float32),
    )(x, wt, b)


def _rnn(g_seq, h0p, lenp, whh_t, bhh, blk=2048):
    Mt, Wp, D = g_seq.shape

    def body(g_ref, h0_ref, len_ref, whh_ref, bhh_ref, out_ref, h_scr):
        t = pl.program_id(1)

        @pl.when(t == 0)
        def _():
            h_scr[...] = h0_ref[...]

        h = h_scr[...]
        hn = jnp.tanh(g_ref[0] + jnp.dot(h, whh_ref[...],
                                         preferred_element_type=jnp.float32)
                      + bhh_ref[...])
        mask = t < len_ref[...]
        h_scr[...] = jnp.where(mask, hn, h)

        @pl.when(t == Mt - 1)
        def _():
            out_ref[...] = h_scr[...]

    return pl.pallas_call(
        body,
        grid=(Wp // blk, Mt),
        in_specs=[pl.BlockSpec((1, blk, D), lambda w, t: (t, w, 0)),
                  pl.BlockSpec((blk, D), lambda w, t: (w, 0)),
                  pl.BlockSpec((blk, 1), lambda w, t: (w, 0)),
                  pl.BlockSpec((D, D), lambda w, t: (0, 0)),
                  pl.BlockSpec((1, D), lambda w, t: (0, 0))],
        out_specs=pl.BlockSpec((blk, D), lambda w, t: (w, 0)),
        out_shape=jax.ShapeDtypeStruct((Wp, D), jnp.float32),
        scratch_shapes=[pltpu.VMEM((blk, D), jnp.float32)],
        compiler_params=pltpu.CompilerParams(
            dimension_semantics=("parallel", "arbitrary")),
    )(g_seq, h0p, lenp, whh_t, bhh)


def _mlp(xp, w1t, g1, be1, w2t, g2, be2, w3t, b3, n_valid, blk=2048):
    Fp, K = xp.shape
    H = w3t.shape[1]

    def body(x_ref, w1_ref, g1_ref, be1_ref, w2_ref, g2_ref, be2_ref,
             w3_ref, b3_ref, o_ref):
        i = pl.program_id(0)
        h1 = _ln_relu(jnp.dot(x_ref[...], w1_ref[...],
                              preferred_element_type=jnp.float32),
                      g1_ref[...], be1_ref[...])
        h2 = _ln_relu(jnp.dot(h1, w2_ref[...],
                              preferred_element_type=jnp.float32),
                      g2_ref[...], be2_ref[...])
        p = jnp.dot(h2, w3_ref[...],
                    preferred_element_type=jnp.float32) + b3_ref[...]
        rows = i * blk + lax.broadcasted_iota(jnp.int32, (blk, 1), 0)
        o_ref[...] = jnp.where(rows < n_valid, p, 0.0)

    return pl.pallas_call(
        body,
        grid=(Fp // blk,),
        in_specs=[pl.BlockSpec((blk, K), lambda i: (i, 0)),
                  pl.BlockSpec((K, 512), lambda i: (0, 0)),
                  pl.BlockSpec((1, 512), lambda i: (0, 0)),
                  pl.BlockSpec((1, 512), lambda i: (0, 0)),
                  pl.BlockSpec((512, 256), lambda i: (0, 0)),
                  pl.BlockSpec((1, 256), lambda i: (0, 0)),
                  pl.BlockSpec((1, 256), lambda i: (0, 0)),
                  pl.BlockSpec((256, H), lambda i: (0, 0)),
                  pl.BlockSpec((1, H), lambda i: (0, 0))],
        out_specs=pl.BlockSpec((blk, H), lambda i: (i, 0)),
        out_shape=jax.ShapeDtypeStruct((Fp, H), jnp.float32),
    )(xp, w1t, g1, be1, w2t, g2, be2, w3t, b3)


def _final(ff, s2, cnt, w4at, w4ct, b3, g4, be4, blk=2000):
    R, K = ff.shape
    D = w4at.shape[1]

    def body(a_ref, b_ref, c_ref, wa_ref, wc_ref, b3_ref, g_ref, be_ref,
             o_ref):
        y = (jnp.dot(a_ref[...], wa_ref[...],
                     preferred_element_type=jnp.float32)
             + b_ref[...]
             + c_ref[...] * jnp.dot(b3_ref[...], wc_ref[...],
                                    preferred_element_type=jnp.float32))
        o_ref[...] = _ln_relu(y, g_ref[...], be_ref[...])

    return pl.pallas_call(
        body,
        grid=(R // blk,),
        in_specs=[pl.BlockSpec((blk, K), lambda i: (i, 0)),
                  pl.BlockSpec((blk, D), lambda i: (i, 0)),
                  pl.BlockSpec((blk, 1), lambda i: (i, 0)),
                  pl.BlockSpec((K, D), lambda i: (0, 0)),
                  pl.BlockSpec((K, D), lambda i: (0, 0)),
                  pl.BlockSpec((1, K), lambda i: (0, 0)),
                  pl.BlockSpec((1, D), lambda i: (0, 0)),
                  pl.BlockSpec((1, D), lambda i: (0, 0))],
        out_specs=pl.BlockSpec((blk, D), lambda i: (i, 0)),
        out_shape=jax.ShapeDtypeStruct((R, D), jnp.float32),
    )(ff, s2, cnt, w4at, w4ct, b3, g4, be4)


def _sc_gather(table, idx, CH=128, NB=4, split=(40, 40)):
    B = idx.shape[0]
    T, D = table.shape
    C0, C1 = split
    assert NS * (C0 + C1) * CH == B and C0 % NB == 0 and C1 % NB == 0
    IDXLEN = C0 * CH
    mesh = plsc.VectorSubcoreMesh(**_SC_MESH)

    @functools.partial(
        pl.kernel, mesh=mesh,
        out_type=jax.ShapeDtypeStruct((B, D), jnp.float32),
        scratch_types=[pltpu.VMEM((IDXLEN,), jnp.int32),
                       pltpu.VMEM((NB, CH, D), jnp.float32),
                       pltpu.SemaphoreType.DMA,
                       pltpu.SemaphoreType.DMA],
    )
    def k(table_hbm, idxp_hbm, out_hbm, idx_v, rows_v, gsem, wsem):
        cid = lax.axis_index("c")
        sid = lax.axis_index("s")
        chunk0 = jnp.where(cid == 0, sid * C0, NS * C0 + sid * C1)
        nblk = jnp.where(cid == 0, C0 // NB, C1 // NB)
        base = chunk0 * CH
        pltpu.sync_copy(idxp_hbm.at[pl.ds(base, IDXLEN)], idx_v)

        def blk(i, carry):
            c0 = i * NB
            gd, wd = [], []
            for b in range(NB):
                src = table_hbm.at[idx_v.at[pl.ds((c0 + b) * CH, CH)]]
                gd.append(pltpu.async_copy(src, rows_v.at[b], gsem))
            for b in range(NB):
                gd[b].wait()
                dst = out_hbm.at[pl.ds(base + (c0 + b) * CH, CH)]
                wd.append(pltpu.async_copy(rows_v.at[b], dst, wsem))
            for b in range(NB):
                wd[b].wait()
            return carry

        lax.fori_loop(0, nblk, blk, 0)

    return k(table, jnp.pad(idx, (0, IDXLEN)))


def _sc_gather_sum(table, idx, lens, zero_row, spread=2048, GRP=8,
                   split_cores=(40, 40)):
    Tp, D = table.shape
    Fp, M = idx.shape
    NIDX = GRP * M
    NB = 4 if D <= 128 else 2
    C0, C1 = split_cores
    assert NS * (C0 + C1) * GRP == Fp and C0 % NB == 0 and C1 % NB == 0
    FLEN = C0 * GRP
    mesh = plsc.VectorSubcoreMesh(**_SC_MESH)

    @functools.partial(
        pl.kernel, mesh=mesh,
        out_type=jax.ShapeDtypeStruct((Fp, D), jnp.float32),
        scratch_types=[pltpu.VMEM((FLEN + 16,), jnp.int32),
                       pltpu.VMEM((FLEN * M,), jnp.int32),
                       pltpu.VMEM((NB, NIDX, D), jnp.float32),
                       pltpu.VMEM((NB, GRP, D), jnp.float32),
                       pltpu.SemaphoreType.DMA,
                       pltpu.SemaphoreType.DMA],
    )
    def k(table_hbm, idx_hbm, len_hbm, out_hbm, len_v, idx_v, rows_v, acc_v,
          gsem, wsem):
        cid = lax.axis_index("c")
        sid = lax.axis_index("s")
        chunk0 = jnp.where(cid == 0, sid * C0, NS * C0 + sid * C1)
        nblk = jnp.where(cid == 0, C0 // NB, C1 // NB)
        fbase = chunk0 * GRP
        pltpu.sync_copy(len_hbm.at[pl.ds(fbase, FLEN)],
                        len_v.at[pl.ds(0, FLEN)])
        pltpu.sync_copy(idx_hbm.at[pl.ds(fbase * M, FLEN * M)], idx_v)
        lane = lax.broadcasted_iota(jnp.int32, (16,), 0)

        def blk(i, carry):
            c0 = i * NB
            gd, wd = [], []
            for b in range(NB):
                c = c0 + b
                lv = len_v[pl.ds(c * GRP, 16)]
                for j in range(GRP):
                    fl = jnp.full((16,), lv[j], jnp.int32)
                    sl = pl.ds(c * NIDX + j * M, 16)
                    raw = idx_v[sl]
                    pad_row = zero_row + (raw & (spread - 1))
                    idx_v[sl] = jnp.where(lane < fl, raw, pad_row)
                src = table_hbm.at[idx_v.at[pl.ds(c * NIDX, NIDX)]]
                gd.append(pltpu.async_copy(src, rows_v.at[b], gsem))
            for b in range(NB):
                gd[b].wait()
                rv = rows_v.at[b]
                av = acc_v.at[b]

                def face(j, carry2):
                    for db in range(D // 16):
                        cs = pl.ds(db * 16, 16)
                        s = rv[j * M, cs]
                        for m in range(1, M):
                            s = s + rv[j * M + m, cs]
                        av[j, cs] = s
                    return carry2

                lax.fori_loop(0, GRP, face, 0)
                dst = out_hbm.at[pl.ds(fbase + (c0 + b) * GRP, GRP)]
                wd.append(pltpu.async_copy(av, dst, wsem))
            for b in range(NB):
                wd[b].wait()
            return carry

        lax.fori_loop(0, nblk, blk, 0)

    return k(table, jnp.pad(idx.reshape(-1), (0, FLEN * M)),
             jnp.pad(lens, (0, FLEN)))


def _pad_rows(x, n):
    return jnp.pad(x, ((0, n - x.shape[0]),) + ((0, 0),) * (x.ndim - 1))


def kernel(edges, faces, edge_index, wire_index, face_index,
           edge_index_length, wire_index_length, adj_face_index_length,
           h0, W_ih, W_hh, b_ih, b_hh, W1, g1, be1, W2, g2, be2, W3, b3,
           W4, g4, be4):
    NE, D = edges.shape
    NF = faces.shape[0]
    NW, Mt = edge_index.shape[0], edge_index.shape[1]
    PAD = NWORK * 8 * 40
    Wp = PAD
    Fp = PAD

    eproj = _matmul_bias(edges, W_ih.T, b_ih.reshape(1, -1))
    eidx = _pad_rows(edge_index, Wp).T.reshape(-1)
    g_seq = _sc_gather(eproj, eidx).reshape(Mt, Wp, D)
    h0p = _pad_rows(h0, Wp)
    elenp = _pad_rows(edge_index_length.reshape(-1, 1), Wp)
    feat_wire = _rnn(g_seq, h0p, elenp, W_hh.T, b_hh.reshape(1, -1))

    SPREAD = 2048
    Tp = PAD + SPREAD
    widx = _pad_rows(wire_index, Fp)
    wlenp = _pad_rows(wire_index_length.reshape(-1, 1), Fp).reshape(-1)
    s1 = _sc_gather_sum(_pad_rows(feat_wire, Tp), widx, wlenp,
                        zero_row=NW, spread=SPREAD)[:NF]

    ff = jnp.concatenate([faces, s1], axis=-1)

    w4c_t = W4[:, 2 * D:].T
    zb = jnp.zeros((1, D), jnp.float32)
    w34_t = _matmul_bias(W3.T, w4c_t, zb, blk=256)
    p = _mlp(_pad_rows(ff, Fp), W1.T, g1.reshape(1, -1), be1.reshape(1, -1),
             W2.T, g2.reshape(1, -1), be2.reshape(1, -1),
             w34_t, zb, n_valid=NF)

    fidx = _pad_rows(face_index, Fp)
    alenp = _pad_rows(adj_face_index_length.reshape(-1, 1), Fp).reshape(-1)
    s2 = _sc_gather_sum(_pad_rows(p, Tp), fidx, alenp, zero_row=NF,
                        spread=SPREAD)[:NF]

    cnt = adj_face_index_length.astype(jnp.float32).reshape(-1, 1)
    return _final(ff, s2, cnt, W4[:, :2 * D].T, w4c_t,
                  b3.reshape(1, -1), g4.reshape(1, -1), be4.reshape(1, -1))


# speedup vs baseline: 7.7387x; 1.0887x over previous
import functools

import jax
import jax.numpy as jnp
from jax import lax
from jax.experimental import pallas as pl
from jax.experimental.pallas import tpu as pltpu
from jax.experimental.pallas import tpu_sc as plsc

NC, NS = 2, 16
NWORK = NC * NS
_SC_MESH = dict(core_axis_name="c", subcore_axis_name="s",
                num_cores=NC, num_subcores=NS)


def _ln_relu(y, g, b):
    mu = jnp.mean(y, axis=-1, keepdims=True)
    var = jnp.mean((y - mu) ** 2, axis=-1, keepdims=True)
    return jax.nn.relu((y - mu) / jnp.sqrt(var + 1e-5) * g + b)


def _matmul_bias(x, wt, b, blk=2000):
    R, K = x.shape
    N = wt.shape[1]

    def body(x_ref, wt_ref, b_ref, o_ref):
        o_ref[...] = jnp.dot(x_ref[...], wt_ref[...],
                             preferred_element_type=jnp.float32) + b_ref[...]

    return pl.pallas_call(
        body,
        grid=(R // blk,),
        in_specs=[pl.BlockSpec((blk, K), lambda i: (i, 0)),
                  pl.BlockSpec((K, N), lambda i: (0, 0)),
                  pl.BlockSpec((1, N), lambda i: (0, 0))],
        out_specs=pl.BlockSpec((blk, N), lambda i: (i, 0)),
        out_shape=jax.ShapeDtypeStruct((R, N), jnp.float32),
    )(x, wt, b)


def _rnn(g_seq, h0p, lenp, whh_t, bhh, t_off=0, blk=2048):
    Mt, Wp, D = g_seq.shape

    def body(g_ref, h0_ref, len_ref, whh_ref, bhh_ref, out_ref, h_scr):
        t = pl.program_id(1)

        @pl.when(t == 0)
        def _():
            h_scr[...] = h0_ref[...]

        h = h_scr[...]
        hn = jnp.tanh(g_ref[0] + jnp.dot(h, whh_ref[...],
                                         preferred_element_type=jnp.float32)
                      + bhh_ref[...])
        mask = t_off + t < len_ref[...]
        h_scr[...] = jnp.where(mask, hn, h)

        @pl.when(t == Mt - 1)
        def _():
            out_ref[...] = h_scr[...]

    return pl.pallas_call(
        body,
        grid=(Wp // blk, Mt),
        in_specs=[pl.BlockSpec((1, blk, D), lambda w, t: (t, w, 0)),
                  pl.BlockSpec((blk, D), lambda w, t: (w, 0)),
                  pl.BlockSpec((blk, 1), lambda w, t: (w, 0)),
                  pl.BlockSpec((D, D), lambda w, t: (0, 0)),
                  pl.BlockSpec((1, D), lambda w, t: (0, 0))],
        out_specs=pl.BlockSpec((blk, D), lambda w, t: (w, 0)),
        out_shape=jax.ShapeDtypeStruct((Wp, D), jnp.float32),
        scratch_shapes=[pltpu.VMEM((blk, D), jnp.float32)],
        compiler_params=pltpu.CompilerParams(
            dimension_semantics=("parallel", "arbitrary")),
    )(g_seq, h0p, lenp, whh_t, bhh)


def _mlp(xp, w1t, g1, be1, w2t, g2, be2, w3t, b3, n_valid, blk=2048):
    Fp, K = xp.shape
    H = w3t.shape[1]

    def body(x_ref, w1_ref, g1_ref, be1_ref, w2_ref, g2_ref, be2_ref,
             w3_ref, b3_ref, o_ref):
        i = pl.program_id(0)
        h1 = _ln_relu(jnp.dot(x_ref[...], w1_ref[...],
                              preferred_element_type=jnp.float32),
                      g1_ref[...], be1_ref[...])
        h2 = _ln_relu(jnp.dot(h1, w2_ref[...],
                              preferred_element_type=jnp.float32),
                      g2_ref[...], be2_ref[...])
        p = jnp.dot(h2, w3_ref[...],
                    preferred_element_type=jnp.float32) + b3_ref[...]
        rows = i * blk + lax.broadcasted_iota(jnp.int32, (blk, 1), 0)
        o_ref[...] = jnp.where(rows < n_valid, p, 0.0)

    return pl.pallas_call(
        body,
        grid=(Fp // blk,),
        in_specs=[pl.BlockSpec((blk, K), lambda i: (i, 0)),
                  pl.BlockSpec((K, 512), lambda i: (0, 0)),
                  pl.BlockSpec((1, 512), lambda i: (0, 0)),
                  pl.BlockSpec((1, 512), lambda i: (0, 0)),
                  pl.BlockSpec((512, 256), lambda i: (0, 0)),
                  pl.BlockSpec((1, 256), lambda i: (0, 0)),
                  pl.BlockSpec((1, 256), lambda i: (0, 0)),
                  pl.BlockSpec((256, H), lambda i: (0, 0)),
                  pl.BlockSpec((1, H), lambda i: (0, 0))],
        out_specs=pl.BlockSpec((blk, H), lambda i: (i, 0)),
        out_shape=jax.ShapeDtypeStruct((Fp, H), jnp.float32),
    )(xp, w1t, g1, be1, w2t, g2, be2, w3t, b3)


def _final(ff, s2, cnt, w4at, w4ct, b3, g4, be4, blk=2000):
    R, K = ff.shape
    D = w4at.shape[1]

    def body(a_ref, b_ref, c_ref, wa_ref, wc_ref, b3_ref, g_ref, be_ref,
             o_ref):
        y = (jnp.dot(a_ref[...], wa_ref[...],
                     preferred_element_type=jnp.float32)
             + b_ref[...]
             + c_ref[...] * jnp.dot(b3_ref[...], wc_ref[...],
                                    preferred_element_type=jnp.float32))
        o_ref[...] = _ln_relu(y, g_ref[...], be_ref[...])

    return pl.pallas_call(
        body,
        grid=(R // blk,),
        in_specs=[pl.BlockSpec((blk, K), lambda i: (i, 0)),
                  pl.BlockSpec((blk, D), lambda i: (i, 0)),
                  pl.BlockSpec((blk, 1), lambda i: (i, 0)),
                  pl.BlockSpec((K, D), lambda i: (0, 0)),
                  pl.BlockSpec((K, D), lambda i: (0, 0)),
                  pl.BlockSpec((1, K), lambda i: (0, 0)),
                  pl.BlockSpec((1, D), lambda i: (0, 0)),
                  pl.BlockSpec((1, D), lambda i: (0, 0))],
        out_specs=pl.BlockSpec((blk, D), lambda i: (i, 0)),
        out_shape=jax.ShapeDtypeStruct((R, D), jnp.float32),
    )(ff, s2, cnt, w4at, w4ct, b3, g4, be4)


def _sc_gather(table, idx, CH=128, NB=4, split=(40, 40)):
    B = idx.shape[0]
    T, D = table.shape
    C0, C1 = split
    assert NS * (C0 + C1) * CH == B and C0 % NB == 0 and C1 % NB == 0
    IDXLEN = C0 * CH
    mesh = plsc.VectorSubcoreMesh(**_SC_MESH)

    @functools.partial(
        pl.kernel, mesh=mesh,
        out_type=jax.ShapeDtypeStruct((B, D), jnp.float32),
        scratch_types=[pltpu.VMEM((IDXLEN,), jnp.int32),
                       pltpu.VMEM((NB, CH, D), jnp.float32),
                       pltpu.SemaphoreType.DMA,
                       pltpu.SemaphoreType.DMA],
    )
    def k(table_hbm, idxp_hbm, out_hbm, idx_v, rows_v, gsem, wsem):
        cid = lax.axis_index("c")
        sid = lax.axis_index("s")
        chunk0 = jnp.where(cid == 0, sid * C0, NS * C0 + sid * C1)
        nblk = jnp.where(cid == 0, C0 // NB, C1 // NB)
        base = chunk0 * CH
        pltpu.sync_copy(idxp_hbm.at[pl.ds(base, IDXLEN)], idx_v)

        def blk(i, carry):
            c0 = i * NB
            gd, wd = [], []
            for b in range(NB):
                src = table_hbm.at[idx_v.at[pl.ds((c0 + b) * CH, CH)]]
                gd.append(pltpu.async_copy(src, rows_v.at[b], gsem))
            for b in range(NB):
                gd[b].wait()
                dst = out_hbm.at[pl.ds(base + (c0 + b) * CH, CH)]
                wd.append(pltpu.async_copy(rows_v.at[b], dst, wsem))
            for b in range(NB):
                wd[b].wait()
            return carry

        lax.fori_loop(0, nblk, blk, 0)

    return k(table, jnp.pad(idx, (0, IDXLEN)))


def _sc_gather_sum(table, idx, lens, zero_row, spread=2048, GRP=8,
                   split_cores=(40, 40)):
    Tp, D = table.shape
    Fp, M = idx.shape
    NIDX = GRP * M
    NB = 4 if D <= 128 else 2
    C0, C1 = split_cores
    assert NS * (C0 + C1) * GRP == Fp and C0 % NB == 0 and C1 % NB == 0
    FLEN = C0 * GRP
    mesh = plsc.VectorSubcoreMesh(**_SC_MESH)

    @functools.partial(
        pl.kernel, mesh=mesh,
        out_type=jax.ShapeDtypeStruct((Fp, D), jnp.float32),
        scratch_types=[pltpu.VMEM((FLEN + 16,), jnp.int32),
                       pltpu.VMEM((FLEN * M,), jnp.int32),
                       pltpu.VMEM((NB, NIDX, D), jnp.float32),
                       pltpu.VMEM((NB, GRP, D), jnp.float32),
                       pltpu.SemaphoreType.DMA,
                       pltpu.SemaphoreType.DMA],
    )
    def k(table_hbm, idx_hbm, len_hbm, out_hbm, len_v, idx_v, rows_v, acc_v,
          gsem, wsem):
        cid = lax.axis_index("c")
        sid = lax.axis_index("s")
        chunk0 = jnp.where(cid == 0, sid * C0, NS * C0 + sid * C1)
        nblk = jnp.where(cid == 0, C0 // NB, C1 // NB)
        fbase = chunk0 * GRP
        pltpu.sync_copy(len_hbm.at[pl.ds(fbase, FLEN)],
                        len_v.at[pl.ds(0, FLEN)])
        pltpu.sync_copy(idx_hbm.at[pl.ds(fbase * M, FLEN * M)], idx_v)
        lane = lax.broadcasted_iota(jnp.int32, (16,), 0)

        def blk(i, carry):
            c0 = i * NB
            gd, wd = [], []
            for b in range(NB):
                c = c0 + b
                lv = len_v[pl.ds(c * GRP, 16)]
                for j in range(GRP):
                    fl = jnp.full((16,), lv[j], jnp.int32)
                    sl = pl.ds(c * NIDX + j * M, 16)
                    raw = idx_v[sl]
                    pad_row = zero_row + (raw & (spread - 1))
                    idx_v[sl] = jnp.where(lane < fl, raw, pad_row)
                src = table_hbm.at[idx_v.at[pl.ds(c * NIDX, NIDX)]]
                gd.append(pltpu.async_copy(src, rows_v.at[b], gsem))
            for b in range(NB):
                gd[b].wait()
                rv = rows_v.at[b]
                av = acc_v.at[b]

                def face(j, carry2):
                    for db in range(D // 16):
                        cs = pl.ds(db * 16, 16)
                        s = rv[j * M, cs]
                        for m in range(1, M):
                            s = s + rv[j * M + m, cs]
                        av[j, cs] = s
                    return carry2

                lax.fori_loop(0, GRP, face, 0)
                dst = out_hbm.at[pl.ds(fbase + (c0 + b) * GRP, GRP)]
                wd.append(pltpu.async_copy(av, dst, wsem))
            for b in range(NB):
                wd[b].wait()
            return carry

        lax.fori_loop(0, nblk, blk, 0)

    return k(table, jnp.pad(idx.reshape(-1), (0, FLEN * M)),
             jnp.pad(lens, (0, FLEN)))


def _pad_rows(x, n):
    return jnp.pad(x, ((0, n - x.shape[0]),) + ((0, 0),) * (x.ndim - 1))


def kernel(edges, faces, edge_index, wire_index, face_index,
           edge_index_length, wire_index_length, adj_face_index_length,
           h0, W_ih, W_hh, b_ih, b_hh, W1, g1, be1, W2, g2, be2, W3, b3,
           W4, g4, be4):
    NE, D = edges.shape
    NF = faces.shape[0]
    NW, Mt = edge_index.shape[0], edge_index.shape[1]
    PAD = NWORK * 8 * 40
    Wp = PAD
    Fp = PAD

    eproj = _matmul_bias(edges, W_ih.T, b_ih.reshape(1, -1))
    Mh = Mt // 2
    eidx = _pad_rows(edge_index, Wp).T.reshape(Mt, Wp)
    whh_t = W_hh.T
    bhh = b_hh.reshape(1, -1)
    h0p = _pad_rows(h0, Wp)
    elenp = _pad_rows(edge_index_length.reshape(-1, 1), Wp)
    g_a = _sc_gather(eproj, eidx[:Mh].reshape(-1),
                     split=(20, 20)).reshape(Mh, Wp, D)
    g_b = _sc_gather(eproj, eidx[Mh:].reshape(-1),
                     split=(20, 20)).reshape(Mh, Wp, D)
    h_mid = _rnn(g_a, h0p, elenp, whh_t, bhh, t_off=0)
    feat_wire = _rnn(g_b, h_mid, elenp, whh_t, bhh, t_off=Mh)

    SPREAD = 2048
    Tp = PAD + SPREAD
    widx = _pad_rows(wire_index, Fp)
    wlenp = _pad_rows(wire_index_length.reshape(-1, 1), Fp).reshape(-1)
    s1 = _sc_gather_sum(_pad_rows(feat_wire, Tp), widx, wlenp,
                        zero_row=NW, spread=SPREAD)[:NF]

    ff = jnp.concatenate([faces, s1], axis=-1)

    w4c_t = W4[:, 2 * D:].T
    zb = jnp.zeros((1, D), jnp.float32)
    w34_t = _matmul_bias(W3.T, w4c_t, zb, blk=256)
    p = _mlp(_pad_rows(ff, Fp), W1.T, g1.reshape(1, -1), be1.reshape(1, -1),
             W2.T, g2.reshape(1, -1), be2.reshape(1, -1),
             w34_t, zb, n_valid=NF)

    fidx = _pad_rows(face_index, Fp)
    alenp = _pad_rows(adj_face_index_length.reshape(-1, 1), Fp).reshape(-1)
    s2 = _sc_gather_sum(_pad_rows(p, Tp), fidx, alenp, zero_row=NF,
                        spread=SPREAD)[:NF]

    cnt = adj_face_index_length.astype(jnp.float32).reshape(-1, 1)
    return _final(ff, s2, cnt, W4[:, :2 * D].T, w4c_t,
                  b3.reshape(1, -1), g4.reshape(1, -1), be4.reshape(1, -1))


# speedup vs baseline: 8.5712x; 1.1076x over previous
import functools

import jax
import jax.numpy as jnp
from jax import lax
from jax.experimental import pallas as pl
from jax.experimental.pallas import tpu as pltpu
from jax.experimental.pallas import tpu_sc as plsc

NC, NS = 2, 16
NWORK = NC * NS
_SC_MESH = dict(core_axis_name="c", subcore_axis_name="s",
                num_cores=NC, num_subcores=NS)


def _ln_relu(y, g, b):
    mu = jnp.mean(y, axis=-1, keepdims=True)
    var = jnp.mean((y - mu) ** 2, axis=-1, keepdims=True)
    return jax.nn.relu((y - mu) / jnp.sqrt(var + 1e-5) * g + b)


def _matmul_bias(x, wt, b, blk=2000):
    R, K = x.shape
    N = wt.shape[1]

    def body(x_ref, wt_ref, b_ref, o_ref):
        o_ref[...] = jnp.dot(x_ref[...], wt_ref[...],
                             preferred_element_type=jnp.float32) + b_ref[...]

    return pl.pallas_call(
        body,
        grid=(R // blk,),
        in_specs=[pl.BlockSpec((blk, K), lambda i: (i, 0)),
                  pl.BlockSpec((K, N), lambda i: (0, 0)),
                  pl.BlockSpec((1, N), lambda i: (0, 0))],
        out_specs=pl.BlockSpec((blk, N), lambda i: (i, 0)),
        out_shape=jax.ShapeDtypeStruct((R, N), jnp.float32),
    )(x, wt, b)


def _rnn(g_seq, h0, lens, whh_t, bhh, t_off=0, out_rows=None, blk=2000):
    Mt = g_seq.shape[0]
    Wr, D = h0.shape
    nb = Wr // blk
    out_rows = out_rows or Wr
    nbo = out_rows // blk

    def body(g_ref, h0_ref, len_ref, whh_ref, bhh_ref, out_ref, h_scr):
        w = pl.program_id(0)
        t = pl.program_id(1)

        @pl.when(w < nb)
        def _():
            @pl.when(t == 0)
            def _():
                h_scr[...] = h0_ref[...]

            h = h_scr[...]
            hn = jnp.tanh(g_ref[0]
                          + jnp.dot(h, whh_ref[...],
                                    preferred_element_type=jnp.float32)
                          + bhh_ref[...])
            mask = t_off + t < len_ref[...]
            h_scr[...] = jnp.where(mask, hn, h)

            @pl.when(t == Mt - 1)
            def _():
                out_ref[...] = h_scr[...]

        @pl.when((w >= nb) & (t == Mt - 1))
        def _():
            out_ref[...] = jnp.zeros_like(out_ref)

    clamp = lambda w: jnp.minimum(w, nb - 1)
    return pl.pallas_call(
        body,
        grid=(nbo, Mt),
        in_specs=[pl.BlockSpec((1, blk, D), lambda w, t: (t, clamp(w), 0)),
                  pl.BlockSpec((blk, D), lambda w, t: (clamp(w), 0)),
                  pl.BlockSpec((blk, 1), lambda w, t: (clamp(w), 0)),
                  pl.BlockSpec((D, D), lambda w, t: (0, 0)),
                  pl.BlockSpec((1, D), lambda w, t: (0, 0))],
        out_specs=pl.BlockSpec((blk, D), lambda w, t: (w, 0)),
        out_shape=jax.ShapeDtypeStruct((out_rows, D), jnp.float32),
        scratch_shapes=[pltpu.VMEM((blk, D), jnp.float32)],
        compiler_params=pltpu.CompilerParams(
            dimension_semantics=("parallel", "arbitrary")),
    )(g_seq, h0, lens, whh_t, bhh)


def _mlp(faces, s1, w1at, w1bt, g1, be1, w2t, g2, be2, w3t, b3, out_rows,
         blk=2000):
    Fr, D = faces.shape
    H = w3t.shape[1]
    nb = Fr // blk
    nbo = out_rows // blk

    def body(a_ref, b_ref, w1a_ref, w1b_ref, g1_ref, be1_ref, w2_ref,
             g2_ref, be2_ref, w3_ref, b3_ref, o_ref):
        i = pl.program_id(0)

        @pl.when(i < nb)
        def _():
            y1 = (jnp.dot(a_ref[...], w1a_ref[...],
                          preferred_element_type=jnp.float32)
                  + jnp.dot(b_ref[...], w1b_ref[...],
                            preferred_element_type=jnp.float32))
            h1 = _ln_relu(y1, g1_ref[...], be1_ref[...])
            h2 = _ln_relu(jnp.dot(h1, w2_ref[...],
                                  preferred_element_type=jnp.float32),
                          g2_ref[...], be2_ref[...])
            o_ref[...] = jnp.dot(h2, w3_ref[...],
                                 preferred_element_type=jnp.float32) + b3_ref[...]

        @pl.when(i >= nb)
        def _():
            o_ref[...] = jnp.zeros_like(o_ref)

    clamp = lambda i: jnp.minimum(i, nb - 1)
    return pl.pallas_call(
        body,
        grid=(nbo,),
        in_specs=[pl.BlockSpec((blk, D), lambda i: (clamp(i), 0)),
                  pl.BlockSpec((blk, D), lambda i: (clamp(i), 0)),
                  pl.BlockSpec((D, 512), lambda i: (0, 0)),
                  pl.BlockSpec((D, 512), lambda i: (0, 0)),
                  pl.BlockSpec((1, 512), lambda i: (0, 0)),
                  pl.BlockSpec((1, 512), lambda i: (0, 0)),
                  pl.BlockSpec((512, 256), lambda i: (0, 0)),
                  pl.BlockSpec((1, 256), lambda i: (0, 0)),
                  pl.BlockSpec((1, 256), lambda i: (0, 0)),
                  pl.BlockSpec((256, H), lambda i: (0, 0)),
                  pl.BlockSpec((1, H), lambda i: (0, 0))],
        out_specs=pl.BlockSpec((blk, H), lambda i: (i, 0)),
        out_shape=jax.ShapeDtypeStruct((out_rows, H), jnp.float32),
    )(faces, s1, w1at, w1bt, g1, be1, w2t, g2, be2, w3t, b3)


def _final(faces, s1, s2, cnt, w4at, w4bt, w4ct, b3, g4, be4, blk=2000):
    R, D = faces.shape
    H = w4at.shape[1]

    def body(a_ref, s1_ref, s2_ref, c_ref, wa_ref, wb_ref, wc_ref, b3_ref,
             g_ref, be_ref, o_ref):
        y = (jnp.dot(a_ref[...], wa_ref[...],
                     preferred_element_type=jnp.float32)
             + jnp.dot(s1_ref[...], wb_ref[...],
                       preferred_element_type=jnp.float32)
             + s2_ref[...]
             + c_ref[...] * jnp.dot(b3_ref[...], wc_ref[...],
                                    preferred_element_type=jnp.float32))
        o_ref[...] = _ln_relu(y, g_ref[...], be_ref[...])

    return pl.pallas_call(
        body,
        grid=(R // blk,),
        in_specs=[pl.BlockSpec((blk, D), lambda i: (i, 0)),
                  pl.BlockSpec((blk, D), lambda i: (i, 0)),
                  pl.BlockSpec((blk, H), lambda i: (i, 0)),
                  pl.BlockSpec((blk, 1), lambda i: (i, 0)),
                  pl.BlockSpec((D, H), lambda i: (0, 0)),
                  pl.BlockSpec((D, H), lambda i: (0, 0)),
                  pl.BlockSpec((256, H), lambda i: (0, 0)),
                  pl.BlockSpec((1, 256), lambda i: (0, 0)),
                  pl.BlockSpec((1, H), lambda i: (0, 0)),
                  pl.BlockSpec((1, H), lambda i: (0, 0))],
        out_specs=pl.BlockSpec((blk, H), lambda i: (i, 0)),
        out_shape=jax.ShapeDtypeStruct((R, H), jnp.float32),
    )(faces, s1, s2, cnt, w4at, w4bt, w4ct, b3, g4, be4)


def _sc_gather(table, idx, CH=128, NB=4, split=(40, 40)):
    B = idx.shape[0]
    T, D = table.shape
    C0, C1 = split
    assert NS * (C0 + C1) * CH == B and C0 % NB == 0 and C1 % NB == 0
    IDXLEN = C0 * CH
    mesh = plsc.VectorSubcoreMesh(**_SC_MESH)

    @functools.partial(
        pl.kernel, mesh=mesh,
        out_type=jax.ShapeDtypeStruct((B, D), jnp.float32),
        scratch_types=[pltpu.VMEM((IDXLEN,), jnp.int32),
                       pltpu.VMEM((NB, CH, D), jnp.float32)]
                      + [pltpu.SemaphoreType.DMA] * (2 * NB),
    )
    def k(table_hbm, idxp_hbm, out_hbm, idx_v, rows_v, *sems):
        gsems, wsems = sems[:NB], sems[NB:]
        cid = lax.axis_index("c")
        sid = lax.axis_index("s")
        chunk0 = jnp.where(cid == 0, sid * C0, NS * C0 + sid * C1)
        nblk = jnp.where(cid == 0, C0 // NB, C1 // NB)
        base = chunk0 * CH
        pltpu.sync_copy(idxp_hbm.at[pl.ds(base, IDXLEN)], idx_v)

        def blk(i, carry):
            c0 = i * NB
            for b in range(NB):
                src = table_hbm.at[idx_v.at[pl.ds((c0 + b) * CH, CH)]]
                pltpu.async_copy(src, rows_v.at[b], gsems[b])
            for b in range(NB):
                src = table_hbm.at[idx_v.at[pl.ds((c0 + b) * CH, CH)]]
                pltpu.make_async_copy(src, rows_v.at[b], gsems[b]).wait()
                dst = out_hbm.at[pl.ds(base + (c0 + b) * CH, CH)]
                pltpu.async_copy(rows_v.at[b], dst, wsems[b])
            for b in range(NB):
                dst = out_hbm.at[pl.ds(base + (c0 + b) * CH, CH)]
                pltpu.make_async_copy(rows_v.at[b], dst, wsems[b]).wait()
            return carry

        lax.fori_loop(0, nblk, blk, 0)

    return k(table, jnp.pad(idx, (0, IDXLEN)))


def _sc_gather_sum(table, idx, lens, zero_row, spread=2048, GRP=8,
                   split_cores=(40, 40)):
    Tp, D = table.shape
    Fp, M = idx.shape
    NIDX = GRP * M
    NB = 4 if D <= 128 else 2
    C0, C1 = split_cores
    assert NS * (C0 + C1) * GRP == Fp and C0 % NB == 0 and C1 % NB == 0
    FLEN = C0 * GRP
    mesh = plsc.VectorSubcoreMesh(**_SC_MESH)

    @functools.partial(
        pl.kernel, mesh=mesh,
        out_type=jax.ShapeDtypeStruct((Fp, D), jnp.float32),
        scratch_types=[pltpu.VMEM((FLEN + 16,), jnp.int32),
                       pltpu.VMEM((FLEN * M,), jnp.int32),
                       pltpu.VMEM((NB, NIDX, D), jnp.float32),
                       pltpu.VMEM((NB, GRP, D), jnp.float32)]
                      + [pltpu.SemaphoreType.DMA] * (2 * NB),
    )
    def k(table_hbm, idx_hbm, len_hbm, out_hbm, len_v, idx_v, rows_v, acc_v,
          *sems):
        gsems, wsems = sems[:NB], sems[NB:]
        cid = lax.axis_index("c")
        sid = lax.axis_index("s")
        chunk0 = jnp.where(cid == 0, sid * C0, NS * C0 + sid * C1)
        n_ch = jnp.where(cid == 0, C0, C1)
        fbase = chunk0 * GRP
        pltpu.sync_copy(len_hbm.at[pl.ds(fbase, FLEN)],
                        len_v.at[pl.ds(0, FLEN)])
        pltpu.sync_copy(idx_hbm.at[pl.ds(fbase * M, FLEN * M)], idx_v)
        lane = lax.broadcasted_iota(jnp.int32, (16,), 0)

        def mask_and_fire(c, b):
            lv = len_v[pl.ds(c * GRP, 16)]
            for j in range(GRP):
                fl = jnp.full((16,), lv[j], jnp.int32)
                sl = pl.ds(c * NIDX + j * M, 16)
                raw = idx_v[sl]
                pad_row = zero_row + (raw & (spread - 1))
                idx_v[sl] = jnp.where(lane < fl, raw, pad_row)
            src = table_hbm.at[idx_v.at[pl.ds(c * NIDX, NIDX)]]
            pltpu.async_copy(src, rows_v.at[b], gsems[b])

        def wait_gather(c, b):
            src = table_hbm.at[idx_v.at[pl.ds(c * NIDX, NIDX)]]
            pltpu.make_async_copy(src, rows_v.at[b], gsems[b]).wait()

        for b in range(NB):
            mask_and_fire(b, b)

        def blk(i, carry):
            for b in range(NB):
                c = i * NB + b
                wait_gather(c, b)

                @pl.when(i > 0)
                def _():
                    dstp = out_hbm.at[pl.ds(fbase + (c - NB) * GRP, GRP)]
                    pltpu.make_async_copy(acc_v.at[b], dstp, wsems[b]).wait()

                rv = rows_v.at[b]
                av = acc_v.at[b]

                def face(j, carry2):
                    for db in range(D // 16):
                        cs = pl.ds(db * 16, 16)
                        ss = rv[j * M, cs]
                        for m in range(1, M):
                            ss = ss + rv[j * M + m, cs]
                        av[j, cs] = ss
                    return carry2

                lax.fori_loop(0, GRP, face, 0)
                dst = out_hbm.at[pl.ds(fbase + c * GRP, GRP)]
                pltpu.async_copy(av, dst, wsems[b])

                @pl.when(c + NB < n_ch)
                def _():
                    mask_and_fire(c + NB, b)

            return carry

        nblk = jnp.where(cid == 0, C0 // NB, C1 // NB)
        lax.fori_loop(0, nblk, blk, 0)
        for b in range(NB):
            last = (nblk - 1) * NB + b
            dst = out_hbm.at[pl.ds(fbase + last * GRP, GRP)]
            pltpu.make_async_copy(acc_v.at[b], dst, wsems[b]).wait()

    return k(table, jnp.pad(idx.reshape(-1), (0, FLEN * M)),
             jnp.pad(lens, (0, FLEN)))


def _pad_rows(x, n):
    return jnp.pad(x, ((0, n - x.shape[0]),) + ((0, 0),) * (x.ndim - 1))


def kernel(edges, faces, edge_index, wire_index, face_index,
           edge_index_length, wire_index_length, adj_face_index_length,
           h0, W_ih, W_hh, b_ih, b_hh, W1, g1, be1, W2, g2, be2, W3, b3,
           W4, g4, be4):
    NE, D = edges.shape
    NF = faces.shape[0]
    NW, Mt = edge_index.shape[0], edge_index.shape[1]
    Wp = Fp = NWORK * 8 * 40
    SPREAD = 2048
    Tp = 14000

    eproj = _matmul_bias(edges, W_ih.T, b_ih.reshape(1, -1))
    Mq = Mt // 2
    eidx = _pad_rows(edge_index, Wp).T.reshape(Mt, Wp)
    whh_t = W_hh.T
    bhh = b_hh.reshape(1, -1)
    elen = edge_index_length.reshape(-1, 1)
    gs = [_sc_gather(eproj, eidx[q * Mq:(q + 1) * Mq].reshape(-1),
                     split=(20, 20)).reshape(Mq, Wp, D)
          for q in range(2)]
    h = _rnn(gs[0], h0, elen, whh_t, bhh, t_off=0)
    feat_wire = _rnn(gs[1], h, elen, whh_t, bhh, t_off=Mq,
                     out_rows=Tp)

    widx = _pad_rows(wire_index, Fp)
    wlenp = _pad_rows(wire_index_length.reshape(-1, 1), Fp).reshape(-1)
    s1 = _sc_gather_sum(feat_wire, widx, wlenp,
                        zero_row=NW, spread=SPREAD)

    w4c_t = W4[:, 2 * D:].T
    zb = jnp.zeros((1, D), jnp.float32)
    w34_t = _matmul_bias(W3.T, w4c_t, zb, blk=256)
    p = _mlp(faces, s1, W1[:, :D].T, W1[:, D:].T,
             g1.reshape(1, -1), be1.reshape(1, -1),
             W2.T, g2.reshape(1, -1), be2.reshape(1, -1),
             w34_t, zb, out_rows=Tp)

    fidx = _pad_rows(face_index, Fp)
    alenp = _pad_rows(adj_face_index_length.reshape(-1, 1), Fp).reshape(-1)
    s2 = _sc_gather_sum(p, fidx, alenp, zero_row=NF,
                        spread=SPREAD)

    cnt = adj_face_index_length.astype(jnp.float32).reshape(-1, 1)
    return _final(faces, s1, s2, cnt, W4[:, :D].T, W4[:, D:2 * D].T,
                  w4c_t, b3.reshape(1, -1), g4.reshape(1, -1),
                  be4.reshape(1, -1))
